# baseline (speedup 1.0000x reference)
import functools

import jax
import jax.numpy as jnp
from jax import lax
from jax.experimental import pallas as pl
from jax.experimental.pallas import tpu as pltpu
from jax.experimental.pallas import tpu_sc as plsc

N = 10000
NPAD = 10240
F = 128
EMB = 64
E = 320000
NC = 2
NS = 16
NW = NC * NS
EW = 128
EPAD = 327680
ER = EPAD // EW
RPT = ER // NW
TRASH = N + 16
ICR = 16
ICH0 = 9
ICH1 = 1
EMBP = 128
NROW_PT = NPAD // NS
NPAD_PT = NPAD // NS

_mesh = plsc.VectorSubcoreMesh(core_axis_name="c", subcore_axis_name="s")


def _wid(cid, sid):
    return cid * NS + sid


@functools.partial(
    pl.kernel,
    out_type=jax.ShapeDtypeStruct((NC * NPAD,), jnp.float32),
    mesh=_mesh,
    scratch_types=[
        pltpu.VMEM_SHARED((NPAD,), jnp.float32),
        pltpu.VMEM((RPT, EW), jnp.int32),
        pltpu.VMEM((NPAD_PT,), jnp.float32),
        pltpu.VMEM((EW,), jnp.float32),
        pltpu.SemaphoreType.DMA,
    ],
)
def _deg_kernel(dst_hbm, out_hbm, acc, dst_v, zbuf, ones_v, ssem):
    cid = lax.axis_index("c")
    sid = lax.axis_index("s")
    wid = _wid(cid, sid)

    def _z(i, _):
        zbuf[pl.ds(i * 16, 16)] = jnp.zeros((16,), jnp.float32)
        return 0
    lax.fori_loop(0, NPAD_PT // 16, _z, 0)
    for k in range(EW // 16):
        ones_v[pl.ds(k * 16, 16)] = jnp.ones((16,), jnp.float32)

    pltpu.sync_copy(zbuf, acc.at[pl.ds(sid * NPAD_PT, NPAD_PT)])
    pltpu.sync_copy(dst_hbm.at[pl.ds(wid * RPT, RPT)], dst_v)
    plsc.subcore_barrier()

    def _body(j, _):
        pltpu.async_copy(ones_v, acc.at[dst_v.at[j]], ssem, add=True)

        @pl.when(j >= 16)
        def _():
            pltpu.make_async_copy(ones_v, acc.at[dst_v.at[j - 16]], ssem).wait()
        return 0
    lax.fori_loop(0, RPT, _body, 0)

    def _drain(j, _):
        pltpu.make_async_copy(ones_v, acc.at[dst_v.at[j]], ssem).wait()
        return 0
    lax.fori_loop(RPT - 16, RPT, _drain, 0)

    plsc.subcore_barrier()
    pltpu.sync_copy(acc.at[pl.ds(sid * NPAD_PT, NPAD_PT)],
                    out_hbm.at[pl.ds(cid * NPAD + sid * NPAD_PT, NPAD_PT)])


@functools.partial(
    pl.kernel,
    out_type=jax.ShapeDtypeStruct((NC, NPAD, EMB), jnp.float32),
    mesh=_mesh,
    scratch_types=[
        pltpu.VMEM_SHARED((NPAD, EMB), jnp.float32),
        pltpu.VMEM((ICR, EW), jnp.int32),
        pltpu.VMEM((ICR, EW), jnp.int32),
        [pltpu.VMEM((EW, EMB), jnp.float32) for _ in range(4)],
        [pltpu.SemaphoreType.DMA for _ in range(4)],
    ],
    compiler_params=pltpu.CompilerParams(use_tc_tiling_on_sc=False),
)
def _gather_add_rows_kernel(src_hbm, dst_hbm, hp_hbm, out_hbm,
                            acc, src_v, dst_v, bufs, sems):
    cid = lax.axis_index("c")
    sid = lax.axis_index("s")

    nch = jnp.where(cid == 0, ICH0, ICH1)
    base = jnp.where(cid == 0, sid * (ICH0 * ICR),
                     NS * ICH0 * ICR + sid * (ICH1 * ICR))

    def _z(i, _):
        for k in range(EMB // 16):
            bufs[0][i, pl.ds(k * 16, 16)] = jnp.zeros((16,), jnp.float32)
        return 0
    lax.fori_loop(0, EW, _z, 0)
    for q in range(NROW_PT // EW):
        pltpu.sync_copy(
            bufs[0], acc.at[pl.ds(sid * NROW_PT + q * EW, EW)])

    plsc.subcore_barrier()

    def _run(hsrc):
        def _chunk(c, _):
            pltpu.sync_copy(src_hbm.at[pl.ds(base + c * ICR, ICR)], src_v)
            pltpu.sync_copy(dst_hbm.at[pl.ds(base + c * ICR, ICR)], dst_v)
            for b in range(4):
                pltpu.async_copy(hsrc.at[src_v.at[b]], bufs[b], sems[b])

            def _body(g, _):
                for b in range(4):
                    j = g * 4 + b
                    pltpu.make_async_copy(hsrc.at[src_v.at[j]],
                                          bufs[b], sems[b]).wait()
                    pltpu.sync_copy(bufs[b], acc.at[dst_v.at[j]], add=True)

                    @pl.when(j + 4 < ICR)
                    def _():
                        pltpu.async_copy(hsrc.at[src_v.at[j + 4]],
                                         bufs[b], sems[b])
                return 0
            lax.fori_loop(0, ICR // 4, _body, 0)
            return 0
        lax.fori_loop(0, nch, _chunk, 0)

    _run(hp_hbm)

    plsc.subcore_barrier()
    pltpu.sync_copy(acc.at[pl.ds(sid * NROW_PT, NROW_PT)],
                    out_hbm.at[cid, pl.ds(sid * NROW_PT, NROW_PT)])


@functools.partial(
    pl.kernel,
    out_type=jax.ShapeDtypeStruct((NC * NPAD,), jnp.float32),
    mesh=_mesh,
    scratch_types=[
        pltpu.VMEM_SHARED((NPAD,), jnp.float32),
        pltpu.VMEM((RPT, EW), jnp.int32),
        pltpu.VMEM((RPT, EW), jnp.int32),
        pltpu.VMEM((N,), jnp.float32),
        pltpu.VMEM((RPT, EW), jnp.float32),
        pltpu.VMEM((NPAD_PT,), jnp.float32),
        pltpu.SemaphoreType.DMA,
    ],
    compiler_params=pltpu.CompilerParams(needs_layout_passes=False),
)
def _gather_add_scalar_kernel(src_hbm, dst_hbm, v_hbm, out_hbm,
                              acc, src_v, dst_v, vfull, data_v, zbuf, ssem):
    cid = lax.axis_index("c")
    sid = lax.axis_index("s")
    wid = _wid(cid, sid)

    def _z(i, _):
        zbuf[pl.ds(i * 16, 16)] = jnp.zeros((16,), jnp.float32)
        return 0
    lax.fori_loop(0, NPAD_PT // 16, _z, 0)
    pltpu.sync_copy(zbuf, acc.at[pl.ds(sid * NPAD_PT, NPAD_PT)])

    pltpu.sync_copy(src_hbm.at[pl.ds(wid * RPT, RPT)], src_v)
    pltpu.sync_copy(dst_hbm.at[pl.ds(wid * RPT, RPT)], dst_v)
    pltpu.sync_copy(v_hbm, vfull)
    plsc.subcore_barrier()

    def _body(j, _):
        for k in range(EW // 16):
            idx = src_v[j, pl.ds(k * 16, 16)]
            data_v[j, pl.ds(k * 16, 16)] = plsc.load_gather(vfull, [idx])
        pltpu.async_copy(data_v.at[j], acc.at[dst_v.at[j]], ssem, add=True)

        @pl.when(j >= 16)
        def _():
            pltpu.make_async_copy(data_v.at[j - 16],
                                  acc.at[dst_v.at[j - 16]], ssem).wait()
        return 0
    lax.fori_loop(0, RPT, _body, 0)

    def _drain(j, _):
        pltpu.make_async_copy(data_v.at[j], acc.at[dst_v.at[j]], ssem).wait()
        return 0
    lax.fori_loop(RPT - 16, RPT, _drain, 0)

    plsc.subcore_barrier()
    pltpu.sync_copy(acc.at[pl.ds(sid * NPAD_PT, NPAD_PT)],
                    out_hbm.at[pl.ds(cid * NPAD + sid * NPAD_PT, NPAD_PT)])


def _b_body(x_ref, w1_ref, degp_ref, hp_ref, dinv_ref):
    degp = degp_ref[...]
    degt = jnp.transpose(degp)[:N, :]
    deg = jnp.sum(degt, axis=1, keepdims=True) + 1.0
    dinv = lax.rsqrt(deg)
    h0 = jnp.dot(x_ref[...], w1_ref[...],
                 preferred_element_type=jnp.float32)
    hp_ref[...] = dinv * h0
    dinv_ref[...] = dinv


def _d_body(p1p_ref, hp_ref, dinv_ref, b1_ref, w2_ref, wm_ref, vp_ref):
    dinv = dinv_ref[...]
    p1 = (p1p_ref[0, :N, :] + p1p_ref[1, :N, :]
          + hp_ref[...])
    h1 = jnp.maximum(dinv * p1 + b1_ref[...][None, :], 0.0)
    w2m = jnp.dot(w2_ref[...], wm_ref[...],
                  preferred_element_type=jnp.float32)
    v = jnp.dot(h1, w2m, preferred_element_type=jnp.float32)
    vp_ref[...] = dinv * v


def _f_body(op_ref, vp_ref, dinv_ref, b2_ref, wm_ref, bm_ref, out_ref):
    op = op_ref[...]
    opt = jnp.transpose(op)[:N, :]
    osum = jnp.sum(opt, axis=1, keepdims=True)
    c = jnp.sum(b2_ref[...][:, None] * wm_ref[...]) + jnp.sum(bm_ref[...])
    out_ref[...] = dinv_ref[...] * (osum + vp_ref[...]) + c


def kernel(x, edge_index, W1, b1, W2, b2, Wm, bm):
    npad_e = EPAD - E
    src = jnp.concatenate(
        [edge_index[0].astype(jnp.int32),
         jnp.zeros((npad_e,), jnp.int32)]).reshape(ER, EW)
    dst = jnp.concatenate(
        [edge_index[1].astype(jnp.int32),
         jnp.full((npad_e,), TRASH, jnp.int32)]).reshape(ER, EW)

    degp = _deg_kernel(dst).reshape(NC, NPAD)

    hp, dinv = pl.pallas_call(
        _b_body,
        out_shape=[jax.ShapeDtypeStruct((N, EMB), jnp.float32),
                   jax.ShapeDtypeStruct((N, 1), jnp.float32)],
    )(x, W1, degp)

    p1p = _gather_add_rows_kernel(src, dst, hp)

    vp = pl.pallas_call(
        _d_body,
        out_shape=jax.ShapeDtypeStruct((N, 1), jnp.float32),
    )(p1p, hp, dinv, b1, W2, Wm)

    op = _gather_add_scalar_kernel(src, dst, vp.reshape(N)).reshape(NC, NPAD)

    logits = pl.pallas_call(
        _f_body,
        out_shape=jax.ShapeDtypeStruct((N, 1), jnp.float32),
    )(op, vp, dinv, b2, Wm, bm)

    return logits.reshape(1, N)


# speedup vs baseline: 1.0302x; 1.0302x over previous
import functools

import jax
import jax.numpy as jnp
from jax import lax
from jax.experimental import pallas as pl
from jax.experimental.pallas import tpu as pltpu
from jax.experimental.pallas import tpu_sc as plsc

N = 10000
NPAD = 10240
F = 128
EMB = 64
E = 320000
NC = 2
NS = 16
NW = NC * NS
EW = 80
EPAD = 327680
ER = EPAD // EW
RPT = ER // NW
TRASH = N + 16
ICR0 = 80
NCH0 = 3
ICR1 = 16
NCH1 = 1
EMBP = 128
NROW_PT = NPAD // NS
NPAD_PT = NPAD // NS

_mesh = plsc.VectorSubcoreMesh(core_axis_name="c", subcore_axis_name="s")


def _wid(cid, sid):
    return cid * NS + sid


@functools.partial(
    pl.kernel,
    out_type=jax.ShapeDtypeStruct((NC * NPAD,), jnp.float32),
    mesh=_mesh,
    scratch_types=[
        pltpu.VMEM_SHARED((NPAD,), jnp.float32),
        pltpu.VMEM((RPT, EW), jnp.int32),
        pltpu.VMEM((NPAD_PT,), jnp.float32),
        pltpu.VMEM((EW,), jnp.float32),
        pltpu.SemaphoreType.DMA,
    ],
)
def _deg_kernel(dst_hbm, out_hbm, acc, dst_v, zbuf, ones_v, ssem):
    cid = lax.axis_index("c")
    sid = lax.axis_index("s")
    wid = _wid(cid, sid)

    def _z(i, _):
        zbuf[pl.ds(i * 16, 16)] = jnp.zeros((16,), jnp.float32)
        return 0
    lax.fori_loop(0, NPAD_PT // 16, _z, 0)
    for k in range(EW // 16):
        ones_v[pl.ds(k * 16, 16)] = jnp.ones((16,), jnp.float32)

    pltpu.sync_copy(zbuf, acc.at[pl.ds(sid * NPAD_PT, NPAD_PT)])
    pltpu.sync_copy(dst_hbm.at[pl.ds(wid * RPT, RPT)], dst_v)
    plsc.subcore_barrier()

    def _body(j, _):
        pltpu.async_copy(ones_v, acc.at[dst_v.at[j]], ssem, add=True)

        @pl.when(j >= 16)
        def _():
            pltpu.make_async_copy(ones_v, acc.at[dst_v.at[j - 16]], ssem).wait()
        return 0
    lax.fori_loop(0, RPT, _body, 0)

    def _drain(j, _):
        pltpu.make_async_copy(ones_v, acc.at[dst_v.at[j]], ssem).wait()
        return 0
    lax.fori_loop(RPT - 16, RPT, _drain, 0)

    plsc.subcore_barrier()
    pltpu.sync_copy(acc.at[pl.ds(sid * NPAD_PT, NPAD_PT)],
                    out_hbm.at[pl.ds(cid * NPAD + sid * NPAD_PT, NPAD_PT)])


@functools.partial(
    pl.kernel,
    out_type=jax.ShapeDtypeStruct((NC, NPAD, EMB), jnp.float32),
    mesh=_mesh,
    scratch_types=[
        pltpu.VMEM_SHARED((NPAD, EMB), jnp.float32),
        pltpu.VMEM((ICR0, EW), jnp.int32),
        pltpu.VMEM((ICR0, EW), jnp.int32),
        [pltpu.VMEM((EW, EMB), jnp.float32) for _ in range(8)],
        [pltpu.SemaphoreType.DMA for _ in range(8)],
        [pltpu.SemaphoreType.DMA for _ in range(8)],
    ],
    compiler_params=pltpu.CompilerParams(use_tc_tiling_on_sc=False),
)
def _gather_add_rows_kernel(src_hbm, dst_hbm, hp_hbm, out_hbm,
                            acc, src_v, dst_v, bufs, gsems, ssems):
    cid = lax.axis_index("c")
    sid = lax.axis_index("s")

    def _z(i, _):
        for k in range(EMB // 16):
            bufs[0][i, pl.ds(k * 16, 16)] = jnp.zeros((16,), jnp.float32)
        return 0
    lax.fori_loop(0, EW, _z, 0)
    for q in range(NROW_PT // EW):
        pltpu.sync_copy(
            bufs[0], acc.at[pl.ds(sid * NROW_PT + q * EW, EW)])

    plsc.subcore_barrier()

    def _run(icr, nch, base):
        def _chunk(c, _):
            pltpu.sync_copy(src_hbm.at[pl.ds(base + c * icr, icr)],
                            src_v.at[pl.ds(0, icr)])
            pltpu.sync_copy(dst_hbm.at[pl.ds(base + c * icr, icr)],
                            dst_v.at[pl.ds(0, icr)])
            for b in range(4):
                pltpu.async_copy(hp_hbm.at[src_v.at[b]], bufs[b], gsems[b])

            def _body(g, _):
                for b in range(8):
                    j = g * 8 + b
                    pltpu.make_async_copy(hp_hbm.at[src_v.at[j]],
                                          bufs[b], gsems[b]).wait()
                    pltpu.async_copy(bufs[b], acc.at[dst_v.at[j]],
                                     ssems[b], add=True)
                    bn = (b + 4) % 8

                    @pl.when(j + 4 < icr)
                    def _():
                        @pl.when(j >= 4)
                        def _():
                            pltpu.make_async_copy(
                                bufs[bn], acc.at[dst_v.at[j - 4]],
                                ssems[bn]).wait()
                        pltpu.async_copy(hp_hbm.at[src_v.at[j + 4]],
                                         bufs[bn], gsems[bn])
                return 0
            lax.fori_loop(0, icr // 8, _body, 0)

            for b in range(8):
                pltpu.make_async_copy(bufs[b], acc.at[dst_v.at[icr - 8 + b]],
                                      ssems[b]).wait()
            return 0
        lax.fori_loop(0, nch, _chunk, 0)

    @pl.when(cid == 0)
    def _():
        _run(ICR0, NCH0, sid * (NCH0 * ICR0))

    @pl.when(cid == 1)
    def _():
        _run(ICR1, NCH1, NS * NCH0 * ICR0 + sid * (NCH1 * ICR1))

    plsc.subcore_barrier()
    pltpu.sync_copy(acc.at[pl.ds(sid * NROW_PT, NROW_PT)],
                    out_hbm.at[cid, pl.ds(sid * NROW_PT, NROW_PT)])


@functools.partial(
    pl.kernel,
    out_type=jax.ShapeDtypeStruct((NC * NPAD,), jnp.float32),
    mesh=_mesh,
    scratch_types=[
        pltpu.VMEM_SHARED((NPAD,), jnp.float32),
        pltpu.VMEM((RPT, EW), jnp.int32),
        pltpu.VMEM((RPT, EW), jnp.int32),
        pltpu.VMEM((N,), jnp.float32),
        pltpu.VMEM((RPT, EW), jnp.float32),
        pltpu.VMEM((NPAD_PT,), jnp.float32),
        pltpu.SemaphoreType.DMA,
    ],
    compiler_params=pltpu.CompilerParams(needs_layout_passes=False),
)
def _gather_add_scalar_kernel(src_hbm, dst_hbm, v_hbm, out_hbm,
                              acc, src_v, dst_v, vfull, data_v, zbuf, ssem):
    cid = lax.axis_index("c")
    sid = lax.axis_index("s")
    wid = _wid(cid, sid)

    def _z(i, _):
        zbuf[pl.ds(i * 16, 16)] = jnp.zeros((16,), jnp.float32)
        return 0
    lax.fori_loop(0, NPAD_PT // 16, _z, 0)
    pltpu.sync_copy(zbuf, acc.at[pl.ds(sid * NPAD_PT, NPAD_PT)])

    pltpu.sync_copy(src_hbm.at[pl.ds(wid * RPT, RPT)], src_v)
    pltpu.sync_copy(dst_hbm.at[pl.ds(wid * RPT, RPT)], dst_v)
    pltpu.sync_copy(v_hbm, vfull)
    plsc.subcore_barrier()

    def _body(j, _):
        for k in range(EW // 16):
            idx = src_v[j, pl.ds(k * 16, 16)]
            data_v[j, pl.ds(k * 16, 16)] = plsc.load_gather(vfull, [idx])
        pltpu.async_copy(data_v.at[j], acc.at[dst_v.at[j]], ssem, add=True)

        @pl.when(j >= 16)
        def _():
            pltpu.make_async_copy(data_v.at[j - 16],
                                  acc.at[dst_v.at[j - 16]], ssem).wait()
        return 0
    lax.fori_loop(0, RPT, _body, 0)

    def _drain(j, _):
        pltpu.make_async_copy(data_v.at[j], acc.at[dst_v.at[j]], ssem).wait()
        return 0
    lax.fori_loop(RPT - 16, RPT, _drain, 0)

    plsc.subcore_barrier()
    pltpu.sync_copy(acc.at[pl.ds(sid * NPAD_PT, NPAD_PT)],
                    out_hbm.at[pl.ds(cid * NPAD + sid * NPAD_PT, NPAD_PT)])


def _b_body(x_ref, w1_ref, degp_ref, hp_ref, dinv_ref):
    degp = degp_ref[...]
    degt = jnp.transpose(degp)[:N, :]
    deg = jnp.sum(degt, axis=1, keepdims=True) + 1.0
    dinv = lax.rsqrt(deg)
    h0 = jnp.dot(x_ref[...], w1_ref[...],
                 preferred_element_type=jnp.float32)
    hp_ref[...] = dinv * h0
    dinv_ref[...] = dinv


def _d_body(p1p_ref, hp_ref, dinv_ref, b1_ref, w2_ref, wm_ref, vp_ref):
    dinv = dinv_ref[...]
    p1 = (p1p_ref[0, :N, :] + p1p_ref[1, :N, :]
          + hp_ref[...])
    h1 = jnp.maximum(dinv * p1 + b1_ref[...][None, :], 0.0)
    w2m = jnp.dot(w2_ref[...], wm_ref[...],
                  preferred_element_type=jnp.float32)
    v = jnp.dot(h1, w2m, preferred_element_type=jnp.float32)
    vp_ref[...] = dinv * v


def _f_body(op_ref, vp_ref, dinv_ref, b2_ref, wm_ref, bm_ref, out_ref):
    op = op_ref[...]
    opt = jnp.transpose(op)[:N, :]
    osum = jnp.sum(opt, axis=1, keepdims=True)
    c = jnp.sum(b2_ref[...][:, None] * wm_ref[...]) + jnp.sum(bm_ref[...])
    out_ref[...] = dinv_ref[...] * (osum + vp_ref[...]) + c


def kernel(x, edge_index, W1, b1, W2, b2, Wm, bm):
    npad_e = EPAD - E
    src = jnp.concatenate(
        [edge_index[0].astype(jnp.int32),
         jnp.zeros((npad_e,), jnp.int32)]).reshape(ER, EW)
    dst = jnp.concatenate(
        [edge_index[1].astype(jnp.int32),
         jnp.full((npad_e,), TRASH, jnp.int32)]).reshape(ER, EW)

    degp = _deg_kernel(dst).reshape(NC, NPAD)

    hp, dinv = pl.pallas_call(
        _b_body,
        out_shape=[jax.ShapeDtypeStruct((N, EMB), jnp.float32),
                   jax.ShapeDtypeStruct((N, 1), jnp.float32)],
    )(x, W1, degp)

    p1p = _gather_add_rows_kernel(src, dst, hp)

    vp = pl.pallas_call(
        _d_body,
        out_shape=jax.ShapeDtypeStruct((N, 1), jnp.float32),
    )(p1p, hp, dinv, b1, W2, Wm)

    op = _gather_add_scalar_kernel(src, dst, vp.reshape(N)).reshape(NC, NPAD)

    logits = pl.pallas_call(
        _f_body,
        out_shape=jax.ShapeDtypeStruct((N, 1), jnp.float32),
    )(op, vp, dinv, b2, Wm, bm)

    return logits.reshape(1, N)
